# baseline (speedup 1.0000x reference)
import functools

import jax
import jax.numpy as jnp
from jax import lax
from jax.experimental import pallas as pl
from jax.experimental.pallas import tpu as pltpu
from jax.experimental.pallas import tpu_sc as plsc

VOCAB = 1000000
EMB = 64
HID = 128
B = 1024
T = 200

NW = 32
N_PER_W = (B * T) // NW
CHUNK = 128
N_CHUNKS = N_PER_W // CHUNK

CW = 1024
NP2 = ((VOCAB + CW - 1) // CW) * (CW // 2)


def _repack_body(in_ref, out_ref):
    x = in_ref[...]
    left = x[:, :CW // 2].T
    right = x[:, CW // 2:].T
    out_ref[...] = jnp.concatenate([left, right], axis=1)


def _repack_table(tableT):
    grid = pl.cdiv(VOCAB, CW)
    return pl.pallas_call(
        _repack_body,
        grid=(grid,),
        in_specs=[pl.BlockSpec((EMB, CW), lambda q: (0, q))],
        out_specs=pl.BlockSpec((CW // 2, 2 * EMB), lambda q: (q, 0)),
        out_shape=jax.ShapeDtypeStruct((NP2, 2 * EMB), jnp.float32),
    )(tableT)


def _sc_gather(table2, idx2):
    mesh = plsc.VectorSubcoreMesh(core_axis_name="c", subcore_axis_name="s")

    @functools.partial(
        pl.kernel,
        out_type=jax.ShapeDtypeStruct((B * T, 2 * EMB), jnp.float32),
        mesh=mesh,
        scratch_types=[
            pltpu.VMEM((N_PER_W,), jnp.int32),
            pltpu.VMEM((CHUNK, 2 * EMB), jnp.float32),
            pltpu.SemaphoreType.DMA,
        ],
    )
    def gather_kernel(table_hbm, idx_hbm, out_hbm, idx_v, rows_v, sem):
        wid = lax.axis_index("s") * 2 + lax.axis_index("c")
        base = wid * N_PER_W
        pltpu.sync_copy(idx_hbm.at[pl.ds(base, N_PER_W)], idx_v)

        def body(g, carry):
            off = pl.multiple_of(g * CHUNK, CHUNK)
            pltpu.async_copy(
                table_hbm.at[idx_v.at[pl.ds(off, CHUNK)]], rows_v, sem
            ).wait()
            pltpu.sync_copy(rows_v, out_hbm.at[pl.ds(base + off, CHUNK)])
            return carry

        lax.fori_loop(0, N_CHUNKS, body, 0)

    return gather_kernel(table2, idx2)


TSTEP = 8


def _gru_body(x_ref, par_ref, wx_ref, wh_ref, b_ref, out_ref, h_ref):
    t = pl.program_id(0)

    @pl.when(t == 0)
    def _init():
        h_ref[...] = jnp.zeros_like(h_ref)

    h = h_ref[...]
    wh = wh_ref[...].astype(jnp.bfloat16)
    wx = wx_ref[...].astype(jnp.bfloat16)
    bias = b_ref[...]
    for j in range(TSTEP):
        xw = x_ref[j]
        par = par_ref[j]
        x = jnp.where(par > 0, xw[:, EMB:], xw[:, :EMB])
        gx = jnp.dot(x.astype(jnp.bfloat16), wx,
                     preferred_element_type=jnp.float32) + bias
        gh = jnp.dot(h.astype(jnp.bfloat16), wh,
                     preferred_element_type=jnp.float32)
        z = jax.nn.sigmoid(gx[:, :HID] + gh[:, :HID])
        r = jax.nn.sigmoid(gx[:, HID:2 * HID] + gh[:, HID:2 * HID])
        cand = jnp.tanh(gx[:, 2 * HID:] + r * gh[:, 2 * HID:])
        h = z * h + (1.0 - z) * cand
        out_ref[:, j, :] = h
    h_ref[...] = h


def _gru(x_tm, par_tm, Wx, Wh, b2):
    return pl.pallas_call(
        _gru_body,
        grid=(T // TSTEP,),
        in_specs=[
            pl.BlockSpec((TSTEP, B, 2 * EMB), lambda t: (t, 0, 0)),
            pl.BlockSpec((TSTEP, B, 1), lambda t: (t, 0, 0)),
            pl.BlockSpec((EMB, 3 * HID), lambda t: (0, 0)),
            pl.BlockSpec((HID, 3 * HID), lambda t: (0, 0)),
            pl.BlockSpec((1, 3 * HID), lambda t: (0, 0)),
        ],
        out_specs=pl.BlockSpec((B, TSTEP, HID), lambda t: (0, t, 0)),
        out_shape=jax.ShapeDtypeStruct((B, T, HID), jnp.float32),
        scratch_shapes=[pltpu.VMEM((B, HID), jnp.float32)],
        compiler_params=pltpu.CompilerParams(
            dimension_semantics=("arbitrary",)
        ),
    )(x_tm, par_tm, Wx, Wh, b2)


def kernel(x_in, seq_lengths, emb_table, Wx, Wh, b):
    del seq_lengths
    idx = x_in.astype(jnp.int32).T.reshape(-1)
    table2 = _repack_table(emb_table.T)
    idx2 = ((idx >> 10) << 9) + (idx & 511)
    half = (idx >> 9) & 1
    xw = _sc_gather(table2, idx2)
    x_tm = xw.reshape(T, B, 2 * EMB)
    par_tm = half.reshape(T, B, 1)
    return _gru(x_tm, par_tm, Wx, Wh, b.reshape(1, 3 * HID))


# speedup vs baseline: 1.7443x; 1.7443x over previous
import functools

import jax
import jax.numpy as jnp
from jax import lax
from jax.experimental import pallas as pl
from jax.experimental.pallas import tpu as pltpu
from jax.experimental.pallas import tpu_sc as plsc

VOCAB = 1000000
EMB = 64
HID = 128
B = 1024
T = 200

NW = 32
N_PER_W = (B * T) // NW
CHUNK = 128
N_CHUNKS = N_PER_W // CHUNK

CW = 8192
NP2 = ((VOCAB + CW - 1) // CW) * (CW // 2)


def _repack_body(in_ref, out_ref):
    x = in_ref[...]
    left = x[:, :CW // 2].T
    right = x[:, CW // 2:].T
    out_ref[...] = jnp.concatenate([left, right], axis=1)


def _repack_table(tableT):
    grid = pl.cdiv(VOCAB, CW)
    return pl.pallas_call(
        _repack_body,
        grid=(grid,),
        in_specs=[pl.BlockSpec((EMB, CW), lambda q: (0, q))],
        out_specs=pl.BlockSpec((CW // 2, 2 * EMB), lambda q: (q, 0)),
        out_shape=jax.ShapeDtypeStruct((NP2, 2 * EMB), jnp.float32),
    )(tableT)


def _sc_gather(table2, idx2):
    mesh = plsc.VectorSubcoreMesh(core_axis_name="c", subcore_axis_name="s")

    @functools.partial(
        pl.kernel,
        out_type=jax.ShapeDtypeStruct((B * T, 2 * EMB), jnp.float32),
        mesh=mesh,
        scratch_types=[
            pltpu.VMEM((N_PER_W,), jnp.int32),
            pltpu.VMEM((CHUNK, 2 * EMB), jnp.float32),
            pltpu.SemaphoreType.DMA,
        ],
    )
    def gather_kernel(table_hbm, idx_hbm, out_hbm, idx_v, rows_v, sem):
        wid = lax.axis_index("s") * 2 + lax.axis_index("c")
        base = wid * N_PER_W
        pltpu.sync_copy(idx_hbm.at[pl.ds(base, N_PER_W)], idx_v)

        def body(g, carry):
            off = pl.multiple_of(g * CHUNK, CHUNK)
            pltpu.async_copy(
                table_hbm.at[idx_v.at[pl.ds(off, CHUNK)]], rows_v, sem
            ).wait()
            pltpu.sync_copy(rows_v, out_hbm.at[pl.ds(base + off, CHUNK)])
            return carry

        lax.fori_loop(0, N_CHUNKS, body, 0)

    return gather_kernel(table2, idx2)


TSTEP = 8


def _gru_body(x_ref, par_ref, wx_ref, wh_ref, b_ref, out_ref, h_ref):
    t = pl.program_id(0)

    @pl.when(t == 0)
    def _init():
        h_ref[...] = jnp.zeros_like(h_ref)

    h = h_ref[...]
    wh = wh_ref[...].astype(jnp.bfloat16)
    wx = wx_ref[...].astype(jnp.bfloat16)
    bias = b_ref[...]
    for j in range(TSTEP):
        xw = x_ref[j]
        par = par_ref[j]
        x = jnp.where(par > 0, xw[:, EMB:], xw[:, :EMB])
        gx = jnp.dot(x.astype(jnp.bfloat16), wx,
                     preferred_element_type=jnp.float32) + bias
        gh = jnp.dot(h.astype(jnp.bfloat16), wh,
                     preferred_element_type=jnp.float32)
        z = jax.nn.sigmoid(gx[:, :HID] + gh[:, :HID])
        r = jax.nn.sigmoid(gx[:, HID:2 * HID] + gh[:, HID:2 * HID])
        cand = jnp.tanh(gx[:, 2 * HID:] + r * gh[:, 2 * HID:])
        h = z * h + (1.0 - z) * cand
        out_ref[:, j, :] = h
    h_ref[...] = h


def _gru(x_tm, par_tm, Wx, Wh, b2):
    return pl.pallas_call(
        _gru_body,
        grid=(T // TSTEP,),
        in_specs=[
            pl.BlockSpec((TSTEP, B, 2 * EMB), lambda t: (t, 0, 0)),
            pl.BlockSpec((TSTEP, B, 1), lambda t: (t, 0, 0)),
            pl.BlockSpec((EMB, 3 * HID), lambda t: (0, 0)),
            pl.BlockSpec((HID, 3 * HID), lambda t: (0, 0)),
            pl.BlockSpec((1, 3 * HID), lambda t: (0, 0)),
        ],
        out_specs=pl.BlockSpec((B, TSTEP, HID), lambda t: (0, t, 0)),
        out_shape=jax.ShapeDtypeStruct((B, T, HID), jnp.float32),
        scratch_shapes=[pltpu.VMEM((B, HID), jnp.float32)],
        compiler_params=pltpu.CompilerParams(
            dimension_semantics=("arbitrary",)
        ),
    )(x_tm, par_tm, Wx, Wh, b2)


def kernel(x_in, seq_lengths, emb_table, Wx, Wh, b):
    del seq_lengths
    idx = x_in.astype(jnp.int32).T.reshape(-1)
    table2 = _repack_table(emb_table.T)
    idx2 = ((idx >> 13) << 12) + (idx & (CW // 2 - 1))
    half = (idx >> 12) & 1
    xw = _sc_gather(table2, idx2)
    x_tm = xw.reshape(T, B, 2 * EMB)
    par_tm = half.reshape(T, B, 1)
    return _gru(x_tm, par_tm, Wx, Wh, b.reshape(1, 3 * HID))


# speedup vs baseline: 1.8260x; 1.0468x over previous
import functools

import jax
import jax.numpy as jnp
from jax import lax
from jax.experimental import pallas as pl
from jax.experimental.pallas import tpu as pltpu
from jax.experimental.pallas import tpu_sc as plsc

VOCAB = 1000000
EMB = 64
HID = 128
B = 1024
T = 200

NW = 32
N_PER_W = (B * T) // NW
CHUNK = 128
N_CHUNKS = N_PER_W // CHUNK

CW = 8192
NP2 = ((VOCAB + CW - 1) // CW) * (CW // 2)


def _repack_body(in_ref, out_ref):
    x = in_ref[...]
    left = x[:, :CW // 2].T
    right = x[:, CW // 2:].T
    out_ref[...] = jnp.concatenate([left, right], axis=1)


def _repack_table(tableT):
    grid = pl.cdiv(VOCAB, CW)
    return pl.pallas_call(
        _repack_body,
        grid=(grid,),
        in_specs=[pl.BlockSpec((EMB, CW), lambda q: (0, q))],
        out_specs=pl.BlockSpec((CW // 2, 2 * EMB), lambda q: (q, 0)),
        out_shape=jax.ShapeDtypeStruct((NP2, 2 * EMB), jnp.float32),
    )(tableT)


def _sc_gather(table2, idx2):
    mesh = plsc.VectorSubcoreMesh(core_axis_name="c", subcore_axis_name="s")

    @functools.partial(
        pl.kernel,
        out_type=jax.ShapeDtypeStruct((B * T, 2 * EMB), jnp.float32),
        mesh=mesh,
        scratch_types=[
            pltpu.VMEM((N_PER_W,), jnp.int32),
            pltpu.VMEM((CHUNK, 2 * EMB), jnp.float32),
            pltpu.SemaphoreType.DMA,
        ],
    )
    def gather_kernel(table_hbm, idx_hbm, out_hbm, idx_v, rows_v, sem):
        wid = lax.axis_index("s") * 2 + lax.axis_index("c")
        base = wid * N_PER_W
        pltpu.sync_copy(idx_hbm.at[pl.ds(base, N_PER_W)], idx_v)

        def body(g, carry):
            off = pl.multiple_of(g * CHUNK, CHUNK)
            pltpu.async_copy(
                table_hbm.at[idx_v.at[pl.ds(off, CHUNK)]], rows_v, sem
            ).wait()
            pltpu.sync_copy(rows_v, out_hbm.at[pl.ds(base + off, CHUNK)])
            return carry

        lax.fori_loop(0, N_CHUNKS, body, 0)

    return gather_kernel(table2, idx2)


TSTEP = 8


def _gru_body(x_ref, par_ref, wx_ref, wh_ref, b_ref, out_ref, h_ref):
    t = pl.program_id(0)

    @pl.when(t == 0)
    def _init():
        h_ref[...] = jnp.zeros_like(h_ref)

    h = h_ref[...]
    wh = wh_ref[...].astype(jnp.bfloat16)
    wx = wx_ref[...].astype(jnp.bfloat16)
    bias = b_ref[...]
    for j in range(TSTEP):
        xw = x_ref[j]
        par = par_ref[j].reshape(B, 1)
        x = jnp.where(par > 0, xw[:, EMB:], xw[:, :EMB])
        gx = jnp.dot(x.astype(jnp.bfloat16), wx,
                     preferred_element_type=jnp.float32) + bias
        gh = jnp.dot(h.astype(jnp.bfloat16), wh,
                     preferred_element_type=jnp.float32)
        z = jax.nn.sigmoid(gx[:, :HID] + gh[:, :HID])
        r = jax.nn.sigmoid(gx[:, HID:2 * HID] + gh[:, HID:2 * HID])
        cand = jnp.tanh(gx[:, 2 * HID:] + r * gh[:, 2 * HID:])
        h = z * h + (1.0 - z) * cand
        out_ref[:, j, :] = h
    h_ref[...] = h


def _gru(x_tm, par_tm, Wx, Wh, b2):
    return pl.pallas_call(
        _gru_body,
        grid=(T // TSTEP,),
        in_specs=[
            pl.BlockSpec((TSTEP, B, 2 * EMB), lambda t: (t, 0, 0)),
            pl.BlockSpec((TSTEP, B), lambda t: (t, 0)),
            pl.BlockSpec((EMB, 3 * HID), lambda t: (0, 0)),
            pl.BlockSpec((HID, 3 * HID), lambda t: (0, 0)),
            pl.BlockSpec((1, 3 * HID), lambda t: (0, 0)),
        ],
        out_specs=pl.BlockSpec((B, TSTEP, HID), lambda t: (0, t, 0)),
        out_shape=jax.ShapeDtypeStruct((B, T, HID), jnp.float32),
        scratch_shapes=[pltpu.VMEM((B, HID), jnp.float32)],
        compiler_params=pltpu.CompilerParams(
            dimension_semantics=("arbitrary",)
        ),
    )(x_tm, par_tm, Wx, Wh, b2)


def kernel(x_in, seq_lengths, emb_table, Wx, Wh, b):
    del seq_lengths
    idx = x_in.astype(jnp.int32).T.reshape(-1)
    table2 = _repack_table(emb_table.T)
    idx2 = ((idx >> 13) << 12) + (idx & (CW // 2 - 1))
    half = (idx >> 12) & 1
    xw = _sc_gather(table2, idx2)
    x_tm = xw.reshape(T, B, 2 * EMB)
    par_tm = half.reshape(T, B)
    return _gru(x_tm, par_tm, Wx, Wh, b.reshape(1, 3 * HID))


# speedup vs baseline: 1.9433x; 1.0643x over previous
import functools

import jax
import jax.numpy as jnp
from jax import lax
from jax.experimental import pallas as pl
from jax.experimental.pallas import tpu as pltpu
from jax.experimental.pallas import tpu_sc as plsc

VOCAB = 1000000
EMB = 64
HID = 128
B = 1024
T = 200

NW = 32
N_PER_W = (B * T) // NW
CHUNK = 128
N_CHUNKS = N_PER_W // CHUNK

CW = 16384
NP2 = ((VOCAB + CW - 1) // CW) * (CW // 2)


def _repack_body(in_ref, out_ref):
    x = in_ref[...]
    left = x[:, :CW // 2].T
    right = x[:, CW // 2:].T
    out_ref[...] = jnp.concatenate([left, right], axis=1)


def _repack_table(tableT):
    grid = pl.cdiv(VOCAB, CW)
    return pl.pallas_call(
        _repack_body,
        grid=(grid,),
        in_specs=[pl.BlockSpec((EMB, CW), lambda q: (0, q))],
        out_specs=pl.BlockSpec((CW // 2, 2 * EMB), lambda q: (q, 0)),
        out_shape=jax.ShapeDtypeStruct((NP2, 2 * EMB), jnp.float32),
    )(tableT)


def _sc_gather(table2, idx2):
    mesh = plsc.VectorSubcoreMesh(core_axis_name="c", subcore_axis_name="s")

    @functools.partial(
        pl.kernel,
        out_type=jax.ShapeDtypeStruct((B * T, 2 * EMB), jnp.float32),
        mesh=mesh,
        scratch_types=[
            pltpu.VMEM((N_PER_W,), jnp.int32),
            pltpu.VMEM((CHUNK, 2 * EMB), jnp.float32),
            pltpu.SemaphoreType.DMA,
        ],
    )
    def gather_kernel(table_hbm, idx_hbm, out_hbm, idx_v, rows_v, sem):
        wid = lax.axis_index("s") * 2 + lax.axis_index("c")
        base = wid * N_PER_W
        pltpu.sync_copy(idx_hbm.at[pl.ds(base, N_PER_W)], idx_v)

        def body(g, carry):
            off = pl.multiple_of(g * CHUNK, CHUNK)
            pltpu.async_copy(
                table_hbm.at[idx_v.at[pl.ds(off, CHUNK)]], rows_v, sem
            ).wait()
            pltpu.sync_copy(rows_v, out_hbm.at[pl.ds(base + off, CHUNK)])
            return carry

        lax.fori_loop(0, N_CHUNKS, body, 0)

    return gather_kernel(table2, idx2)


TSTEP = 8


def _gru_body(x_ref, par_ref, wx_ref, wh_ref, b_ref, out_ref, h_ref):
    t = pl.program_id(0)

    @pl.when(t == 0)
    def _init():
        h_ref[...] = jnp.zeros_like(h_ref)

    h = h_ref[...]
    wh = wh_ref[...].astype(jnp.bfloat16)
    wx = wx_ref[...].astype(jnp.bfloat16)
    bias = b_ref[...]
    for j in range(TSTEP):
        xw = x_ref[j]
        par = par_ref[j].reshape(B, 1)
        x = jnp.where(par > 0, xw[:, EMB:], xw[:, :EMB])
        gx = jnp.dot(x.astype(jnp.bfloat16), wx,
                     preferred_element_type=jnp.float32) + bias
        gh = jnp.dot(h.astype(jnp.bfloat16), wh,
                     preferred_element_type=jnp.float32)
        z = jax.nn.sigmoid(gx[:, :HID] + gh[:, :HID])
        r = jax.nn.sigmoid(gx[:, HID:2 * HID] + gh[:, HID:2 * HID])
        cand = jnp.tanh(gx[:, 2 * HID:] + r * gh[:, 2 * HID:])
        h = z * h + (1.0 - z) * cand
        out_ref[:, j, :] = h
    h_ref[...] = h


def _gru(x_tm, par_tm, Wx, Wh, b2):
    return pl.pallas_call(
        _gru_body,
        grid=(T // TSTEP,),
        in_specs=[
            pl.BlockSpec((TSTEP, B, 2 * EMB), lambda t: (t, 0, 0)),
            pl.BlockSpec((TSTEP, B), lambda t: (t, 0)),
            pl.BlockSpec((EMB, 3 * HID), lambda t: (0, 0)),
            pl.BlockSpec((HID, 3 * HID), lambda t: (0, 0)),
            pl.BlockSpec((1, 3 * HID), lambda t: (0, 0)),
        ],
        out_specs=pl.BlockSpec((B, TSTEP, HID), lambda t: (0, t, 0)),
        out_shape=jax.ShapeDtypeStruct((B, T, HID), jnp.float32),
        scratch_shapes=[pltpu.VMEM((B, HID), jnp.float32)],
        compiler_params=pltpu.CompilerParams(
            dimension_semantics=("arbitrary",)
        ),
    )(x_tm, par_tm, Wx, Wh, b2)


def kernel(x_in, seq_lengths, emb_table, Wx, Wh, b):
    del seq_lengths
    idx = x_in.astype(jnp.int32).T.reshape(-1)
    table2 = _repack_table(emb_table.T)
    idx2 = ((idx >> 14) << 13) + (idx & (CW // 2 - 1))
    half = (idx >> 13) & 1
    xw = _sc_gather(table2, idx2)
    x_tm = xw.reshape(T, B, 2 * EMB)
    par_tm = half.reshape(T, B)
    return _gru(x_tm, par_tm, Wx, Wh, b.reshape(1, 3 * HID))


# speedup vs baseline: 2.0942x; 1.0776x over previous
import functools

import jax
import jax.numpy as jnp
from jax import lax
from jax.experimental import pallas as pl
from jax.experimental.pallas import tpu as pltpu
from jax.experimental.pallas import tpu_sc as plsc

VOCAB = 1000000
EMB = 64
HID = 128
B = 1024
T = 200

NW = 32
N_PER_W = (B * T) // NW
CHUNK = 256
N_CHUNKS = N_PER_W // CHUNK
NBUF = 3

CW = 16384
NP2 = ((VOCAB + CW - 1) // CW) * (CW // 2)


def _repack_body(in_ref, out_ref):
    x = in_ref[...]
    left = x[:, :CW // 2].T
    right = x[:, CW // 2:].T
    out_ref[...] = jnp.concatenate([left, right], axis=1)


def _repack_table(tableT):
    grid = pl.cdiv(VOCAB, CW)
    return pl.pallas_call(
        _repack_body,
        grid=(grid,),
        in_specs=[pl.BlockSpec((EMB, CW), lambda q: (0, q))],
        out_specs=pl.BlockSpec((CW // 2, 2 * EMB), lambda q: (q, 0)),
        out_shape=jax.ShapeDtypeStruct((NP2, 2 * EMB), jnp.float32),
    )(tableT)


def _sc_gather(table2, idx2):
    mesh = plsc.VectorSubcoreMesh(core_axis_name="c", subcore_axis_name="s")

    @functools.partial(
        pl.kernel,
        out_type=jax.ShapeDtypeStruct((B * T, 2 * EMB), jnp.float32),
        mesh=mesh,
        scratch_types=[
            pltpu.VMEM((N_PER_W,), jnp.int32),
            pltpu.VMEM((CHUNK, 2 * EMB), jnp.float32),
            pltpu.VMEM((CHUNK, 2 * EMB), jnp.float32),
            pltpu.VMEM((CHUNK, 2 * EMB), jnp.float32),
            pltpu.SemaphoreType.DMA,
            pltpu.SemaphoreType.DMA,
            pltpu.SemaphoreType.DMA,
            pltpu.SemaphoreType.DMA,
            pltpu.SemaphoreType.DMA,
            pltpu.SemaphoreType.DMA,
        ],
    )
    def gather_kernel(table_hbm, idx_hbm, out_hbm, idx_v,
                      b0, b1, b2, gs0, gs1, gs2, os0, os1, os2):
        bufs = (b0, b1, b2)
        gsems = (gs0, gs1, gs2)
        osems = (os0, os1, os2)
        wid = lax.axis_index("s") * 2 + lax.axis_index("c")
        base = wid * N_PER_W
        pltpu.sync_copy(idx_hbm.at[pl.ds(base, N_PER_W)], idx_v)

        for b in range(NBUF):
            pltpu.async_copy(
                table_hbm.at[idx_v.at[pl.ds(b * CHUNK, CHUNK)]],
                bufs[b], gsems[b])

        def outer(k, carry):
            for b in range(NBUF):
                g = k * NBUF + b

                @pl.when(g < N_CHUNKS)
                def _chunk():
                    pltpu.make_async_copy(
                        table_hbm.at[pl.ds(0, CHUNK)], bufs[b], gsems[b]
                    ).wait()
                    off = pl.multiple_of(g * CHUNK, CHUNK)
                    pltpu.async_copy(
                        bufs[b], out_hbm.at[pl.ds(base + off, CHUNK)],
                        osems[b])

                    @pl.when(g + NBUF < N_CHUNKS)
                    def _refill():
                        pltpu.make_async_copy(
                            bufs[b], out_hbm.at[pl.ds(0, CHUNK)], osems[b]
                        ).wait()
                        off2 = pl.multiple_of((g + NBUF) * CHUNK, CHUNK)
                        pltpu.async_copy(
                            table_hbm.at[idx_v.at[pl.ds(off2, CHUNK)]],
                            bufs[b], gsems[b])
            return carry

        lax.fori_loop(0, (N_CHUNKS + NBUF - 1) // NBUF, outer, 0)

        for b in range(NBUF):
            pltpu.make_async_copy(
                bufs[b], out_hbm.at[pl.ds(0, CHUNK)], osems[b]
            ).wait()

    return gather_kernel(table2, idx2)


TSTEP = 8


def _gru_body(x_ref, par_ref, wx_ref, wh_ref, b_ref, out_ref, h_ref):
    t = pl.program_id(0)

    @pl.when(t == 0)
    def _init():
        h_ref[...] = jnp.zeros_like(h_ref)

    h = h_ref[...]
    wh = wh_ref[...].astype(jnp.bfloat16)
    wx = wx_ref[...].astype(jnp.bfloat16)
    bias = b_ref[...]
    for j in range(TSTEP):
        xw = x_ref[j]
        par = par_ref[j].reshape(B, 1)
        x = jnp.where(par > 0, xw[:, EMB:], xw[:, :EMB])
        gx = jnp.dot(x.astype(jnp.bfloat16), wx,
                     preferred_element_type=jnp.float32) + bias
        gh = jnp.dot(h.astype(jnp.bfloat16), wh,
                     preferred_element_type=jnp.float32)
        z = jax.nn.sigmoid(gx[:, :HID] + gh[:, :HID])
        r = jax.nn.sigmoid(gx[:, HID:2 * HID] + gh[:, HID:2 * HID])
        cand = jnp.tanh(gx[:, 2 * HID:] + r * gh[:, 2 * HID:])
        h = z * h + (1.0 - z) * cand
        out_ref[:, j, :] = h
    h_ref[...] = h


def _gru(x_tm, par_tm, Wx, Wh, b2):
    return pl.pallas_call(
        _gru_body,
        grid=(T // TSTEP,),
        in_specs=[
            pl.BlockSpec((TSTEP, B, 2 * EMB), lambda t: (t, 0, 0)),
            pl.BlockSpec((TSTEP, B), lambda t: (t, 0)),
            pl.BlockSpec((EMB, 3 * HID), lambda t: (0, 0)),
            pl.BlockSpec((HID, 3 * HID), lambda t: (0, 0)),
            pl.BlockSpec((1, 3 * HID), lambda t: (0, 0)),
        ],
        out_specs=pl.BlockSpec((B, TSTEP, HID), lambda t: (0, t, 0)),
        out_shape=jax.ShapeDtypeStruct((B, T, HID), jnp.float32),
        scratch_shapes=[pltpu.VMEM((B, HID), jnp.float32)],
        compiler_params=pltpu.CompilerParams(
            dimension_semantics=("arbitrary",)
        ),
    )(x_tm, par_tm, Wx, Wh, b2)


def kernel(x_in, seq_lengths, emb_table, Wx, Wh, b):
    del seq_lengths
    idx = x_in.astype(jnp.int32).T.reshape(-1)
    table2 = _repack_table(emb_table.T)
    idx2 = ((idx >> 14) << 13) + (idx & (CW // 2 - 1))
    half = (idx >> 13) & 1
    xw = _sc_gather(table2, idx2)
    x_tm = xw.reshape(T, B, 2 * EMB)
    par_tm = half.reshape(T, B)
    return _gru(x_tm, par_tm, Wx, Wh, b.reshape(1, 3 * HID))


# speedup vs baseline: 2.1990x; 1.0501x over previous
import functools

import jax
import jax.numpy as jnp
from jax import lax
from jax.experimental import pallas as pl
from jax.experimental.pallas import tpu as pltpu
from jax.experimental.pallas import tpu_sc as plsc

VOCAB = 1000000
EMB = 64
HID = 128
B = 1024
T = 200

NW = 32
N_PER_W = (B * T) // NW
CHUNK = 256
N_CHUNKS = N_PER_W // CHUNK
NBUF = 3

CW = 16384
SQ = CW // 4
NP2 = ((VOCAB + CW - 1) // CW) * SQ


def _rne_bf16_bits(a):
    u = jax.lax.bitcast_convert_type(a, jnp.uint32)
    return (u + jnp.uint32(0x7FFF) + ((u >> 16) & jnp.uint32(1))) >> 16


def _repack_body(in_ref, out_ref):
    x = in_ref[...]
    s0 = _rne_bf16_bits(x[:, :SQ].T)
    s1 = _rne_bf16_bits(x[:, SQ:2 * SQ].T)
    s2 = _rne_bf16_bits(x[:, 2 * SQ:3 * SQ].T)
    s3 = _rne_bf16_bits(x[:, 3 * SQ:].T)
    p01 = jax.lax.bitcast_convert_type((s1 << 16) | s0, jnp.float32)
    p23 = jax.lax.bitcast_convert_type((s3 << 16) | s2, jnp.float32)
    out_ref[...] = jnp.concatenate([p01, p23], axis=1)


def _repack_table(tableT):
    grid = pl.cdiv(VOCAB, CW)
    return pl.pallas_call(
        _repack_body,
        grid=(grid,),
        in_specs=[pl.BlockSpec((EMB, CW), lambda q: (0, q))],
        out_specs=pl.BlockSpec((SQ, 2 * EMB), lambda q: (q, 0)),
        out_shape=jax.ShapeDtypeStruct((NP2, 2 * EMB), jnp.float32),
    )(tableT)


def _sc_gather(table2, idx2):
    mesh = plsc.VectorSubcoreMesh(core_axis_name="c", subcore_axis_name="s")

    @functools.partial(
        pl.kernel,
        out_type=jax.ShapeDtypeStruct((B * T, 2 * EMB), jnp.float32),
        mesh=mesh,
        scratch_types=[
            pltpu.VMEM((N_PER_W,), jnp.int32),
            pltpu.VMEM((CHUNK, 2 * EMB), jnp.float32),
            pltpu.VMEM((CHUNK, 2 * EMB), jnp.float32),
            pltpu.VMEM((CHUNK, 2 * EMB), jnp.float32),
            pltpu.SemaphoreType.DMA,
            pltpu.SemaphoreType.DMA,
            pltpu.SemaphoreType.DMA,
            pltpu.SemaphoreType.DMA,
            pltpu.SemaphoreType.DMA,
            pltpu.SemaphoreType.DMA,
        ],
    )
    def gather_kernel(table_hbm, idx_hbm, out_hbm, idx_v,
                      b0, b1, b2, gs0, gs1, gs2, os0, os1, os2):
        bufs = (b0, b1, b2)
        gsems = (gs0, gs1, gs2)
        osems = (os0, os1, os2)
        wid = lax.axis_index("s") * 2 + lax.axis_index("c")
        base = wid * N_PER_W
        pltpu.sync_copy(idx_hbm.at[pl.ds(base, N_PER_W)], idx_v)

        for b in range(NBUF):
            pltpu.async_copy(
                table_hbm.at[idx_v.at[pl.ds(b * CHUNK, CHUNK)]],
                bufs[b], gsems[b])

        def outer(k, carry):
            for b in range(NBUF):
                g = k * NBUF + b

                @pl.when(g < N_CHUNKS)
                def _chunk():
                    pltpu.make_async_copy(
                        table_hbm.at[pl.ds(0, CHUNK)], bufs[b], gsems[b]
                    ).wait()
                    off = pl.multiple_of(g * CHUNK, CHUNK)
                    pltpu.async_copy(
                        bufs[b], out_hbm.at[pl.ds(base + off, CHUNK)],
                        osems[b])

                    @pl.when(g + NBUF < N_CHUNKS)
                    def _refill():
                        pltpu.make_async_copy(
                            bufs[b], out_hbm.at[pl.ds(0, CHUNK)], osems[b]
                        ).wait()
                        off2 = pl.multiple_of((g + NBUF) * CHUNK, CHUNK)
                        pltpu.async_copy(
                            table_hbm.at[idx_v.at[pl.ds(off2, CHUNK)]],
                            bufs[b], gsems[b])
            return carry

        lax.fori_loop(0, (N_CHUNKS + NBUF - 1) // NBUF, outer, 0)

        for b in range(NBUF):
            pltpu.make_async_copy(
                bufs[b], out_hbm.at[pl.ds(0, CHUNK)], osems[b]
            ).wait()

    return gather_kernel(table2, idx2)


TSTEP = 8


def _gru_body(x_ref, par_ref, wx_ref, wh_ref, b_ref, out_ref, h_ref):
    t = pl.program_id(0)

    @pl.when(t == 0)
    def _init():
        h_ref[...] = jnp.zeros_like(h_ref)

    h = h_ref[...]
    wh = wh_ref[...].astype(jnp.bfloat16)
    wx = wx_ref[...].astype(jnp.bfloat16)
    bias = b_ref[...]
    for j in range(TSTEP):
        xw = x_ref[j]
        sel = par_ref[j].reshape(B, 1)
        u = jax.lax.bitcast_convert_type(xw, jnp.uint32)
        u64 = jnp.where(sel >= 2, u[:, EMB:], u[:, :EMB])
        xbits = jnp.where((sel & 1) > 0,
                          u64 & jnp.uint32(0xFFFF0000), u64 << 16)
        x = jax.lax.bitcast_convert_type(xbits, jnp.float32)
        gx = jnp.dot(x.astype(jnp.bfloat16), wx,
                     preferred_element_type=jnp.float32) + bias
        gh = jnp.dot(h.astype(jnp.bfloat16), wh,
                     preferred_element_type=jnp.float32)
        z = jax.nn.sigmoid(gx[:, :HID] + gh[:, :HID])
        r = jax.nn.sigmoid(gx[:, HID:2 * HID] + gh[:, HID:2 * HID])
        cand = jnp.tanh(gx[:, 2 * HID:] + r * gh[:, 2 * HID:])
        h = z * h + (1.0 - z) * cand
        out_ref[:, j, :] = h
    h_ref[...] = h


def _gru(x_tm, par_tm, Wx, Wh, b2):
    return pl.pallas_call(
        _gru_body,
        grid=(T // TSTEP,),
        in_specs=[
            pl.BlockSpec((TSTEP, B, 2 * EMB), lambda t: (t, 0, 0)),
            pl.BlockSpec((TSTEP, B), lambda t: (t, 0)),
            pl.BlockSpec((EMB, 3 * HID), lambda t: (0, 0)),
            pl.BlockSpec((HID, 3 * HID), lambda t: (0, 0)),
            pl.BlockSpec((1, 3 * HID), lambda t: (0, 0)),
        ],
        out_specs=pl.BlockSpec((B, TSTEP, HID), lambda t: (0, t, 0)),
        out_shape=jax.ShapeDtypeStruct((B, T, HID), jnp.float32),
        scratch_shapes=[pltpu.VMEM((B, HID), jnp.float32)],
        compiler_params=pltpu.CompilerParams(
            dimension_semantics=("arbitrary",)
        ),
    )(x_tm, par_tm, Wx, Wh, b2)


def kernel(x_in, seq_lengths, emb_table, Wx, Wh, b):
    del seq_lengths
    idx = x_in.astype(jnp.int32).T.reshape(-1)
    table2 = _repack_table(emb_table.T)
    idx2 = ((idx >> 14) << 12) + (idx & (SQ - 1))
    half = (idx >> 12) & 3
    xw = _sc_gather(table2, idx2)
    x_tm = xw.reshape(T, B, 2 * EMB)
    par_tm = half.reshape(T, B)
    return _gru(x_tm, par_tm, Wx, Wh, b.reshape(1, 3 * HID))
